# baseline (speedup 1.0000x reference)
import functools

import jax
import jax.numpy as jnp
import numpy as np
from jax import lax
from jax.experimental import pallas as pl
from jax.experimental.pallas import tpu as pltpu
from jax.experimental.pallas import tpu_sc as plsc

LEVELS = 4
BATCH = 128
D = 1024
HALF = D // 2
NPAIR = 8128
P_EXP = 2

NC = 2
NS = 16
NW = NC * NS
LANES = 16

C = 16
CPL = NPAIR // C
NOUTER = (CPL + 2 * NW - 1) // (2 * NW)

_i0, _i1 = np.triu_indices(n=BATCH, k=1)
NPAD = NPAIR + LANES
I0 = np.zeros(NPAD, np.int32)
I1 = np.zeros(NPAD, np.int32)
I0[:NPAIR] = _i0
I1[:NPAIR] = _i1

_mesh = plsc.VectorSubcoreMesh(core_axis_name="c", subcore_axis_name="s")


@functools.partial(
    pl.kernel,
    mesh=_mesh,
    out_type=[jax.ShapeDtypeStruct((NPAIR, D), jnp.float32)
              for _ in range(LEVELS)],
    scratch_types=[
        pltpu.VMEM((NPAD,), jnp.int32),
        pltpu.VMEM((NPAD,), jnp.int32),
        pltpu.VMEM((BATCH, HALF), jnp.float32),
        pltpu.VMEM((C, HALF), jnp.float32),
        pltpu.VMEM((C, HALF), jnp.float32),
        pltpu.SemaphoreType.DMA,
        pltpu.SemaphoreType.DMA,
    ],
)
def _batch_diff_sc(table_hbm, i0_hbm, i1_hbm,
                   out0, out1, out2, out3,
                   idxa0, idxa1, tab, oba, obb, soa, sob):
    wid = lax.axis_index("s") * NC + lax.axis_index("c")
    outs = (out0, out1, out2, out3)
    ob = (oba, obb)
    so = (soa, sob)

    pltpu.sync_copy(i0_hbm, idxa0)
    pltpu.sync_copy(i1_hbm, idxa1)

    nr = (CPL - 1 - wid) // NW + 1

    for l in range(LEVELS):
        out_l = outs[l]
        for h in range(2):
            pltpu.sync_copy(
                table_hbm.at[pl.ds(l * BATCH, BATCH),
                             pl.ds(h * HALF, HALF)], tab)

            def outer_body(g, _, out_l=out_l, h=h):
                for b in range(2):
                    t = 2 * g + b

                    @pl.when(t < nr)
                    def _(t=t, b=b):
                        chunk = t * NW + wid
                        base = chunk * C

                        @pl.when(t >= 2)
                        def _():
                            pltpu.make_async_copy(
                                ob[b],
                                out_l.at[pl.ds(0, C), pl.ds(h * HALF, HALF)],
                                so[b]).wait()

                        def row_body(r, carry):
                            r0 = idxa0[pl.ds(base + r, LANES)][0]
                            r1 = idxa1[pl.ds(base + r, LANES)][0]
                            for cc in range(HALF // LANES):
                                sl = pl.ds(cc * LANES, LANES)
                                d = tab[r0, sl] - tab[r1, sl]
                                ob[b][r, sl] = d * d
                            return carry

                        lax.fori_loop(0, C, row_body, 0)
                        pltpu.async_copy(
                            ob[b],
                            out_l.at[pl.ds(chunk * C, C),
                                     pl.ds(h * HALF, HALF)],
                            so[b])

                return 0

            lax.fori_loop(0, NOUTER, outer_body, 0)

            for b in range(2):
                pltpu.make_async_copy(
                    ob[b], out_l.at[pl.ds(0, C), pl.ds(h * HALF, HALF)],
                    so[b]).wait()


def kernel(pyramid):
    table = pyramid.reshape(LEVELS * BATCH, D)
    return tuple(_batch_diff_sc(table, I0, I1))


# speedup vs baseline: 1.6734x; 1.6734x over previous
import functools

import jax
import jax.numpy as jnp
import numpy as np
from jax import lax
from jax.experimental import pallas as pl
from jax.experimental.pallas import tpu as pltpu
from jax.experimental.pallas import tpu_sc as plsc

LEVELS = 4
BATCH = 128
D = 1024
NPAIR = 8128
P_EXP = 2

NC = 2
NS = 16
NW = NC * NS
LANES = 16
CR = 16
SEC = 512
NSEC = D // SEC

_mesh = plsc.VectorSubcoreMesh(core_axis_name="c", subcore_axis_name="s")


@functools.partial(
    pl.kernel,
    mesh=_mesh,
    compiler_params=pltpu.CompilerParams(use_tc_tiling_on_sc=False),
    out_type=[jax.ShapeDtypeStruct((NPAIR, D), jnp.float32)
              for _ in range(LEVELS)],
    scratch_types=[
        pltpu.VMEM_SHARED((LEVELS * BATCH, D), jnp.float32),
        pltpu.VMEM((1, D), jnp.float32),
        pltpu.VMEM((CR, D), jnp.float32),
        pltpu.VMEM((CR, D), jnp.float32),
        pltpu.VMEM((CR, D), jnp.float32),
        pltpu.VMEM((CR, D), jnp.float32),
        pltpu.SemaphoreType.DMA,
        pltpu.SemaphoreType.DMA,
        pltpu.SemaphoreType.DMA,
    ],
)
def _batch_diff_sc(table_hbm, out0, out1, out2, out3,
                   tabsp, arow, rj, oba, obb, ob16, swa, swb, st):
    sid = lax.axis_index("s")
    cid = lax.axis_index("c")
    wid = sid * NC + cid
    outs = (out0, out1, out2, out3)
    ob = (oba, obb)
    sw = (swa, swb)

    @pl.when(sid == 0)
    def _():
        pltpu.sync_copy(table_hbm, tabsp)

    plsc.subcore_barrier()

    def run_body(i, out_l, lbase):
        rlen = BATCH - 1 - i
        off_i = i * (BATCH - 1) - (i * (i - 1)) // 2

        @pl.when(rlen >= CR)
        def _():
            pltpu.sync_copy(tabsp.at[pl.ds(lbase + i, 1)], arow)
            nchunk = (rlen + CR - 1) // CR

            def chunk_pair(g, _):
                for b in range(2):
                    k = 2 * g + b

                    @pl.when(k < nchunk)
                    def _(k=k, b=b):
                        start = jnp.minimum(k * CR, rlen - CR)
                        pltpu.sync_copy(
                            tabsp.at[pl.ds(lbase + i + 1 + start, CR)], rj)

                        @pl.when(k >= 2)
                        def _():
                            pltpu.make_async_copy(
                                ob[b], out_l.at[pl.ds(0, CR)], sw[b]).wait()

                        for sec in range(NSEC):
                            a_reg = [arow[0, pl.ds(sec * SEC + m * LANES,
                                                   LANES)]
                                     for m in range(SEC // LANES)]

                            def row_body(r, carry, sec=sec, a_reg=a_reg):
                                for m in range(SEC // LANES):
                                    sl = pl.ds(sec * SEC + m * LANES, LANES)
                                    d = a_reg[m] - rj[r, sl]
                                    ob[b][r, sl] = d * d
                                return carry

                            lax.fori_loop(0, CR, row_body, 0)

                        pltpu.async_copy(
                            ob[b], out_l.at[pl.ds(off_i + start, CR)], sw[b])

                return 0

            lax.fori_loop(0, (nchunk + 1) // 2, chunk_pair, 0)

            pltpu.make_async_copy(ob[0], out_l.at[pl.ds(0, CR)],
                                  sw[0]).wait()

            @pl.when(nchunk >= 2)
            def _():
                pltpu.make_async_copy(ob[1], out_l.at[pl.ds(0, CR)],
                                      sw[1]).wait()

        @pl.when(rlen < CR)
        def _():
            pltpu.sync_copy(tabsp.at[pl.ds(lbase + BATCH - CR, CR)], rj)
            wbase = i - (BATCH - CR)

            def srow_body(r, carry):
                for m in range(D // LANES):
                    sl = pl.ds(m * LANES, LANES)
                    d = rj[wbase, sl] - rj[wbase + 1 + r, sl]
                    ob16[r, sl] = d * d
                return carry

            lax.fori_loop(0, rlen, srow_body, 0)

            def swrite_body(r, carry):
                pltpu.async_copy(ob16.at[pl.ds(r, 1)],
                                 out_l.at[pl.ds(off_i + r, 1)], st)
                return carry

            lax.fori_loop(0, rlen, swrite_body, 0)

            def sdrain_body(r, carry):
                pltpu.make_async_copy(ob16.at[pl.ds(0, 1)],
                                      out_l.at[pl.ds(0, 1)], st).wait()
                return carry

            lax.fori_loop(0, rlen, sdrain_body, 0)

    for l in range(LEVELS):
        out_l = outs[l]
        lbase = l * BATCH

        def s_body(s, _, out_l=out_l, lbase=lbase):
            p = wid + NW * s
            run_body(p, out_l, lbase)
            run_body(BATCH - 2 - p, out_l, lbase)
            return 0

        lax.fori_loop(0, 2, s_body, 0)


def kernel(pyramid):
    table = pyramid.reshape(LEVELS * BATCH, D)
    return tuple(_batch_diff_sc(table))
